# baseline (speedup 1.0000x reference)
import functools

import jax
import jax.numpy as jnp
from jax.experimental import pallas as pl
from jax.experimental.pallas import tpu as pltpu


def _bq_matmul_kernel(x_hbm, bin_hbm, scale_hbm, bias_hbm, out_hbm,
                      x_v, bin_v, scale_v, bias_v, out_v,
                      sem_x, sem_b0, sem_b1, sem_sc, sem_bi, sem_o0, sem_o1):
    nbits, g, nf = bin_v.shape
    h = nf // 2
    c_sc = pltpu.make_async_copy(scale_hbm, scale_v, sem_sc)
    c_sc.start()
    c_b0 = pltpu.make_async_copy(bin_hbm.at[:, :, 0:h], bin_v.at[:, :, 0:h], sem_b0)
    c_b0.start()
    c_x = pltpu.make_async_copy(x_hbm, x_v, sem_x)
    c_x.start()
    c_b1 = pltpu.make_async_copy(bin_hbm.at[:, :, h:nf], bin_v.at[:, :, h:nf], sem_b1)
    c_b1.start()
    c_bi = pltpu.make_async_copy(bias_hbm, bias_v, sem_bi)
    c_bi.start()

    shifts = 7 - jax.lax.broadcasted_iota(jnp.int32, (1, 8, 1), 1)

    def decode(c0):
        acc = None
        csum = None
        for b in range(nbits):
            byte = bin_v[b, :, c0:c0 + h]
            bits = (byte[:, None, :] >> shifts) & 1
            s = scale_v[b, :, c0:c0 + h]
            fb = bits.astype(jnp.float32) * s[None]
            acc = fb if acc is None else acc + fb
            csum = s if csum is None else csum + s
        w = 2.0 * acc - csum[None]
        return w.reshape(g * 8, h).astype(jnp.bfloat16)

    c_sc.wait()
    c_b0.wait()
    w0 = decode(0)
    c_b1.wait()
    w1 = decode(h)
    c_x.wait()
    xb = x_v[...].astype(jnp.bfloat16)
    c_bi.wait()
    o0 = jnp.dot(xb, w0, preferred_element_type=jnp.float32)
    out_v[:, 0:h] = o0 + bias_v[:, 0:h]
    c_o0 = pltpu.make_async_copy(out_v.at[:, 0:h], out_hbm.at[:, 0:h], sem_o0)
    c_o0.start()
    o1 = jnp.dot(xb, w1, preferred_element_type=jnp.float32)
    out_v[:, h:nf] = o1 + bias_v[:, h:nf]
    c_o1 = pltpu.make_async_copy(out_v.at[:, h:nf], out_hbm.at[:, h:nf], sem_o1)
    c_o1.start()
    c_o0.wait()
    c_o1.wait()


@functools.partial(jax.jit, static_argnames=())
def kernel(x, binary, scale, bias):
    size_out = x.shape[:-1] + (bias.shape[-1],)
    x2 = x.reshape(-1, x.shape[-1])
    t, nx = x2.shape
    nbits = scale.shape[1]
    nf = scale.shape[2]
    g = nx // 8
    binary3 = binary.reshape(nbits, g, nf)
    scale3 = scale.reshape(nbits, 1, nf)
    bias2 = bias.reshape(1, nf)
    hbm = pl.BlockSpec(memory_space=pltpu.MemorySpace.HBM)
    out = pl.pallas_call(
        _bq_matmul_kernel,
        in_specs=[hbm, hbm, hbm, hbm],
        out_specs=hbm,
        out_shape=jax.ShapeDtypeStruct((t, nf), jnp.float32),
        scratch_shapes=[
            pltpu.VMEM((t, nx), jnp.float32),
            pltpu.VMEM((nbits, g, nf), jnp.int32),
            pltpu.VMEM((nbits, 1, nf), jnp.float32),
            pltpu.VMEM((1, nf), jnp.float32),
            pltpu.VMEM((t, nf), jnp.float32),
        ] + [pltpu.SemaphoreType.DMA] * 7,
    )(x2, binary3, scale3, bias2)
    return out.reshape(size_out)
